# baseline (speedup 1.0000x reference)
import functools

import jax
import jax.numpy as jnp
from jax import lax
from jax.experimental import pallas as pl
from jax.experimental.pallas import tpu as pltpu
from jax.experimental.pallas import tpu_sc as plsc

D = 300
L = 50
NC, NS = 2, 16
NW = NC * NS
DP = 384
NSEG = 3
RPAD = 112
NCHUNK = 19


def _sc_body(idx_hbm, emb_hbm, wck_hbm, out_hbm, idx_v, wck_v, out_v,
             ent_a, ent_b, dst_a, dst_b, sem_a, sem_b):
    wid = lax.axis_index("c") * NS + lax.axis_index("s")
    iters = idx_v.shape[0]
    row0 = wid * iters

    pltpu.sync_copy(idx_hbm.at[pl.ds(row0, iters)], idx_v)
    pltpu.sync_copy(wck_hbm, wck_v)
    iota = lax.iota(jnp.int32, 16)

    def build(it, ent):
        for m in range(7):
            rcl = jnp.minimum(iota + (16 * m), 2 * L - 1)
            v = plsc.load_gather(idx_v, [jnp.full((16,), it, jnp.int32),
                                         rcl])
            e3 = v * NSEG
            for ct in range(NSEG):
                ent[ct, pl.ds(16 * m, 16)] = e3 + ct

    def start(ent, dst, sem):
        for ct in range(NSEG):
            pltpu.async_copy(emb_hbm.at[ent.at[ct]],
                             dst.at[pl.ds(RPAD * ct, RPAD)], sem)

    def wait(ent, dst, sem):
        for ct in range(NSEG):
            pltpu.make_async_copy(emb_hbm.at[ent.at[ct]],
                                  dst.at[pl.ds(RPAD * ct, RPAD)], sem).wait()

    def compute(dst, it):
        def loads(rg):
            return tuple(
                dst[RPAD * ((16 * c) // 128) + rg, pl.ds((16 * c) % 128, 16)]
                for c in range(NCHUNK))

        for half in range(2):
            base = half * L

            def body(r, accs):
                return tuple(jnp.maximum(a, x)
                             for a, x in zip(accs, loads(base + r)))

            accs = lax.fori_loop(1, L, body, loads(base))
            psum = jnp.zeros((16,), jnp.float32)
            for c in range(NCHUNK):
                psum = psum + accs[c] * wck_v[c, :]
            out_v[2 * it + half, :] = psum

    build(0, ent_a)
    start(ent_a, dst_a, sem_a)

    def step2(k2, _):
        it0 = 2 * k2
        it1 = 2 * k2 + 1
        build(it1, ent_b)
        start(ent_b, dst_b, sem_b)
        wait(ent_a, dst_a, sem_a)
        compute(dst_a, it0)
        it2 = jnp.minimum(it0 + 2, iters - 1)
        build(it2, ent_a)
        start(ent_a, dst_a, sem_a)
        wait(ent_b, dst_b, sem_b)
        compute(dst_b, it1)
        return _

    lax.fori_loop(0, iters // 2, step2, 0)
    wait(ent_a, dst_a, sem_a)
    pltpu.sync_copy(out_v, out_hbm.at[pl.ds(row0 * 2, 2 * iters)])


def _make_sc_call(bags, iters):
    mesh = plsc.VectorSubcoreMesh(
        core_axis_name="c", subcore_axis_name="s",
        num_cores=NC, num_subcores=NS)
    return functools.partial(
        pl.kernel, mesh=mesh,
        compiler_params=pltpu.CompilerParams(
            use_tc_tiling_on_sc=False, needs_layout_passes=False),
        out_type=jax.ShapeDtypeStruct((bags, 16), jnp.float32),
        scratch_types=[
            pltpu.VMEM((iters, 2 * L), jnp.int32),
            pltpu.VMEM((NCHUNK, 16), jnp.float32),
            pltpu.VMEM((2 * iters, 16), jnp.float32),
            pltpu.VMEM((NSEG, RPAD), jnp.int32),
            pltpu.VMEM((NSEG, RPAD), jnp.int32),
            pltpu.VMEM((NSEG * RPAD, 128), jnp.float32),
            pltpu.VMEM((NSEG * RPAD, 128), jnp.float32),
            pltpu.SemaphoreType.DMA,
            pltpu.SemaphoreType.DMA,
        ])(_sc_body)


def _seg_body(in_hbm, out_ref, buf, sem):
    i = pl.program_id(0)
    rows = buf.shape[0]
    cp = pltpu.make_async_copy(in_hbm.at[pl.ds(i * rows, rows)], buf, sem)
    cp.start()
    cp.wait()
    x = jnp.pad(buf[:], ((0, 0), (0, DP - D)))
    out_ref[:] = x.reshape(out_ref.shape)


def _seg_table(emb):
    V = emb.shape[0]
    rows = 5000
    return pl.pallas_call(
        _seg_body,
        grid=(V // rows,),
        in_specs=[pl.BlockSpec(memory_space=pl.ANY)],
        out_specs=pl.BlockSpec((NSEG * rows, 128), lambda i: (i, 0)),
        out_shape=jax.ShapeDtypeStruct((NSEG * V, 128), jnp.float32),
        scratch_shapes=[pltpu.VMEM((rows, D), jnp.float32),
                        pltpu.SemaphoreType.DMA],
    )(emb)


def _tc_body(part_ref, lab_ref, b_ref, logits_ref, loss_ref):
    part = part_ref[:]
    s = jnp.sum(part, axis=1, keepdims=True)
    a = s + b_ref[0, 0]
    logits_ref[:] = a
    n = part.shape[0] // 2
    l0 = a[:n]
    l1 = a[n:]
    m = jnp.maximum(l0, l1)
    logz = m + jnp.log(jnp.exp(l0 - m) + jnp.exp(l1 - m))
    ll = jnp.where(lab_ref[:] == 0, l0, l1)
    loss_ref[0, 0] = jnp.mean(logz - ll)


def kernel(warrant0s, warrant1s, label_ids, emb, W, b):
    B = warrant0s.shape[0]
    bags = 2 * B
    iters = bags // NW // 2

    idx = jnp.concatenate(
        [warrant0s.astype(jnp.int32), warrant1s.astype(jnp.int32)],
        axis=0).reshape(B, 2 * L)

    wck = jnp.pad(W.reshape(-1), (0, NCHUNK * 16 - D)).reshape(NCHUNK, 16)

    emb3 = _seg_table(emb)

    part = _make_sc_call(bags, iters)(idx, emb3, wck)

    logits_col, loss_arr = pl.pallas_call(
        _tc_body,
        out_shape=[
            jax.ShapeDtypeStruct((bags, 1), jnp.float32),
            jax.ShapeDtypeStruct((1, 1), jnp.float32),
        ],
        in_specs=[
            pl.BlockSpec(memory_space=pltpu.VMEM),
            pl.BlockSpec(memory_space=pltpu.VMEM),
            pl.BlockSpec(memory_space=pltpu.SMEM),
        ],
        out_specs=[
            pl.BlockSpec(memory_space=pltpu.VMEM),
            pl.BlockSpec(memory_space=pltpu.SMEM),
        ],
    )(part, label_ids.astype(jnp.int32).reshape(B, 1), b.reshape(1, 1))

    logits = jnp.concatenate([logits_col[:B], logits_col[B:]], axis=1)
    return (loss_arr[0, 0], logits)


# speedup vs baseline: 1.1247x; 1.1247x over previous
import functools

import jax
import jax.numpy as jnp
from jax import lax
from jax.experimental import pallas as pl
from jax.experimental.pallas import tpu as pltpu
from jax.experimental.pallas import tpu_sc as plsc

D = 300
L = 50
NC, NS = 2, 16
NW = NC * NS
DP = 384
NSEG = 3
RPAD = 112
NCHUNK = 19


def _sc_body(idx_hbm, emb_hbm, wck_hbm, out_hbm, idx_v, wck_v, out_v,
             ent_a, ent_b, dst_a, dst_b, sem_a, sem_b):
    wid = lax.axis_index("c") * NS + lax.axis_index("s")
    iters = idx_v.shape[0]
    row0 = wid * iters

    pltpu.sync_copy(idx_hbm.at[pl.ds(row0, iters)], idx_v)
    pltpu.sync_copy(wck_hbm, wck_v)
    iota = lax.iota(jnp.int32, 16)

    def build(it, ent):
        for m in range(7):
            rcl = jnp.minimum(iota + (16 * m), 2 * L - 1)
            v = plsc.load_gather(idx_v, [jnp.full((16,), it, jnp.int32),
                                         rcl])
            e3 = v * NSEG
            for ct in range(NSEG):
                ent[ct, pl.ds(16 * m, 16)] = e3 + ct

    def start(ent, dst, sem):
        for ct in range(NSEG):
            pltpu.async_copy(emb_hbm.at[ent.at[ct]],
                             dst.at[pl.ds(RPAD * ct, RPAD)], sem)

    def wait(ent, dst, sem):
        for ct in range(NSEG):
            pltpu.make_async_copy(emb_hbm.at[ent.at[ct]],
                                  dst.at[pl.ds(RPAD * ct, RPAD)], sem).wait()

    def compute(dst, it):
        def loads(rg):
            return tuple(
                dst[RPAD * ((16 * c) // 128) + rg, pl.ds((16 * c) % 128, 16)]
                for c in range(NCHUNK))

        for half in range(2):
            base = half * L

            def body(r, accs):
                return tuple(jnp.maximum(a, x)
                             for a, x in zip(accs, loads(base + r)))

            accs = lax.fori_loop(1, L, body, loads(base))
            psum = jnp.zeros((16,), jnp.float32)
            for c in range(NCHUNK):
                psum = psum + accs[c] * wck_v[c, :]
            out_v[2 * it + half, :] = psum

    build(0, ent_a)
    start(ent_a, dst_a, sem_a)

    def step2(k2, _):
        it0 = 2 * k2
        it1 = 2 * k2 + 1
        build(it1, ent_b)
        start(ent_b, dst_b, sem_b)
        wait(ent_a, dst_a, sem_a)
        compute(dst_a, it0)
        it2 = jnp.minimum(it0 + 2, iters - 1)
        build(it2, ent_a)
        start(ent_a, dst_a, sem_a)
        wait(ent_b, dst_b, sem_b)
        compute(dst_b, it1)
        return _

    lax.fori_loop(0, iters // 2, step2, 0)
    wait(ent_a, dst_a, sem_a)
    pltpu.sync_copy(out_v, out_hbm.at[pl.ds(row0 * 2, 2 * iters)])


def _make_sc_call(bags, iters):
    mesh = plsc.VectorSubcoreMesh(
        core_axis_name="c", subcore_axis_name="s",
        num_cores=NC, num_subcores=NS)
    return functools.partial(
        pl.kernel, mesh=mesh,
        compiler_params=pltpu.CompilerParams(
            use_tc_tiling_on_sc=False, needs_layout_passes=False),
        out_type=jax.ShapeDtypeStruct((bags, 16), jnp.float32),
        scratch_types=[
            pltpu.VMEM((iters, 2 * L), jnp.int32),
            pltpu.VMEM((NCHUNK, 16), jnp.float32),
            pltpu.VMEM((2 * iters, 16), jnp.float32),
            pltpu.VMEM((NSEG, RPAD), jnp.int32),
            pltpu.VMEM((NSEG, RPAD), jnp.int32),
            pltpu.VMEM((NSEG * RPAD, 128), jnp.float32),
            pltpu.VMEM((NSEG * RPAD, 128), jnp.float32),
            pltpu.SemaphoreType.DMA,
            pltpu.SemaphoreType.DMA,
        ])(_sc_body)


def _seg_body(in_ref, out_ref):
    x = jnp.pad(in_ref[:], ((0, 0), (0, DP - D)))
    out_ref[:] = x.reshape(out_ref.shape)


def _seg_table(emb):
    V = emb.shape[0]
    rows = 5000
    return pl.pallas_call(
        _seg_body,
        grid=(V // rows,),
        in_specs=[pl.BlockSpec((rows, D), lambda i: (i, 0))],
        out_specs=pl.BlockSpec((NSEG * rows, 128), lambda i: (i, 0)),
        out_shape=jax.ShapeDtypeStruct((NSEG * V, 128), jnp.float32),
    )(emb)


def _tc_body(part_ref, lab_ref, b_ref, logits_ref, loss_ref):
    part = part_ref[:]
    s = jnp.sum(part, axis=1, keepdims=True)
    a = s + b_ref[0, 0]
    logits_ref[:] = a
    n = part.shape[0] // 2
    l0 = a[:n]
    l1 = a[n:]
    m = jnp.maximum(l0, l1)
    logz = m + jnp.log(jnp.exp(l0 - m) + jnp.exp(l1 - m))
    ll = jnp.where(lab_ref[:] == 0, l0, l1)
    loss_ref[0, 0] = jnp.mean(logz - ll)


def kernel(warrant0s, warrant1s, label_ids, emb, W, b):
    B = warrant0s.shape[0]
    bags = 2 * B
    iters = bags // NW // 2

    idx = jnp.concatenate(
        [warrant0s.astype(jnp.int32), warrant1s.astype(jnp.int32)],
        axis=0).reshape(B, 2 * L)

    wck = jnp.pad(W.reshape(-1), (0, NCHUNK * 16 - D)).reshape(NCHUNK, 16)

    emb3 = _seg_table(emb)

    part = _make_sc_call(bags, iters)(idx, emb3, wck)

    logits_col, loss_arr = pl.pallas_call(
        _tc_body,
        out_shape=[
            jax.ShapeDtypeStruct((bags, 1), jnp.float32),
            jax.ShapeDtypeStruct((1, 1), jnp.float32),
        ],
        in_specs=[
            pl.BlockSpec(memory_space=pltpu.VMEM),
            pl.BlockSpec(memory_space=pltpu.VMEM),
            pl.BlockSpec(memory_space=pltpu.SMEM),
        ],
        out_specs=[
            pl.BlockSpec(memory_space=pltpu.VMEM),
            pl.BlockSpec(memory_space=pltpu.SMEM),
        ],
    )(part, label_ids.astype(jnp.int32).reshape(B, 1), b.reshape(1, 1))

    logits = jnp.concatenate([logits_col[:B], logits_col[B:]], axis=1)
    return (loss_arr[0, 0], logits)


# speedup vs baseline: 1.2347x; 1.0978x over previous
import functools

import jax
import jax.numpy as jnp
from jax import lax
from jax.experimental import pallas as pl
from jax.experimental.pallas import tpu as pltpu
from jax.experimental.pallas import tpu_sc as plsc

D = 300
L = 50
NC, NS = 2, 16
NW = NC * NS
DP = 384
NSEG = 3
RPAD = 104
NCHUNK = 19


def _sc_body(idx_hbm, emb_hbm, wck_hbm, out_hbm, idx_v, wck_v, out_v,
             ent_a, ent_b, dst_a, dst_b, sem_a, sem_b):
    wid = lax.axis_index("c") * NS + lax.axis_index("s")
    iters = idx_v.shape[0]
    row0 = wid * iters

    pltpu.sync_copy(idx_hbm.at[pl.ds(row0, iters)], idx_v)
    pltpu.sync_copy(wck_hbm, wck_v)
    iota = lax.iota(jnp.int32, 16)

    def build(it, ent):
        for m in range(7):
            off = min(16 * m, RPAD - 16)
            rcl = jnp.minimum(iota + off, 2 * L - 1)
            v = plsc.load_gather(idx_v, [jnp.full((16,), it, jnp.int32),
                                         rcl])
            e3 = v * NSEG
            for ct in range(NSEG):
                ent[ct, pl.ds(off, 16)] = e3 + ct

    def start(ent, dst, sem):
        for ct in range(NSEG):
            pltpu.async_copy(emb_hbm.at[ent.at[ct]],
                             dst.at[pl.ds(RPAD * ct, RPAD)], sem)

    def wait(ent, dst, sem):
        for ct in range(NSEG):
            pltpu.make_async_copy(emb_hbm.at[ent.at[ct]],
                                  dst.at[pl.ds(RPAD * ct, RPAD)], sem).wait()

    def compute(dst, it):
        def loads(rg):
            return tuple(
                dst[RPAD * ((16 * c) // 128) + rg, pl.ds((16 * c) % 128, 16)]
                for c in range(NCHUNK))

        for half in range(2):
            base = half * L

            def body(r, accs):
                return tuple(jnp.maximum(a, x)
                             for a, x in zip(accs, loads(base + r)))

            accs = lax.fori_loop(1, L, body, loads(base))
            psum = jnp.zeros((16,), jnp.float32)
            for c in range(NCHUNK):
                psum = psum + accs[c] * wck_v[c, :]
            out_v[2 * it + half, :] = psum

    build(0, ent_a)
    start(ent_a, dst_a, sem_a)

    def step2(k2, _):
        it0 = 2 * k2
        it1 = 2 * k2 + 1
        build(it1, ent_b)
        start(ent_b, dst_b, sem_b)
        wait(ent_a, dst_a, sem_a)
        compute(dst_a, it0)
        it2 = jnp.minimum(it0 + 2, iters - 1)
        build(it2, ent_a)
        start(ent_a, dst_a, sem_a)
        wait(ent_b, dst_b, sem_b)
        compute(dst_b, it1)
        return _

    lax.fori_loop(0, iters // 2, step2, 0)
    wait(ent_a, dst_a, sem_a)
    pltpu.sync_copy(out_v, out_hbm.at[pl.ds(row0 * 2, 2 * iters)])


def _make_sc_call(bags, iters):
    mesh = plsc.VectorSubcoreMesh(
        core_axis_name="c", subcore_axis_name="s",
        num_cores=NC, num_subcores=NS)
    return functools.partial(
        pl.kernel, mesh=mesh,
        compiler_params=pltpu.CompilerParams(
            use_tc_tiling_on_sc=False, needs_layout_passes=False),
        out_type=jax.ShapeDtypeStruct((bags, 16), jnp.float32),
        scratch_types=[
            pltpu.VMEM((iters, 2 * L), jnp.int32),
            pltpu.VMEM((NCHUNK, 16), jnp.float32),
            pltpu.VMEM((2 * iters, 16), jnp.float32),
            pltpu.VMEM((NSEG, RPAD), jnp.int32),
            pltpu.VMEM((NSEG, RPAD), jnp.int32),
            pltpu.VMEM((NSEG * RPAD, 128), jnp.float32),
            pltpu.VMEM((NSEG * RPAD, 128), jnp.float32),
            pltpu.SemaphoreType.DMA,
            pltpu.SemaphoreType.DMA,
        ])(_sc_body)


def _seg_body(in_ref, out_ref):
    x = jnp.pad(in_ref[:], ((0, 0), (0, DP - D)))
    out_ref[:] = x.reshape(out_ref.shape)


def _seg_table(emb):
    V = emb.shape[0]
    rows = 5000
    return pl.pallas_call(
        _seg_body,
        grid=(V // rows,),
        in_specs=[pl.BlockSpec((rows, D), lambda i: (i, 0))],
        out_specs=pl.BlockSpec((NSEG * rows, 128), lambda i: (i, 0)),
        out_shape=jax.ShapeDtypeStruct((NSEG * V, 128), jnp.float32),
    )(emb)


def _tc_body(part_ref, lab_ref, b_ref, logits_ref, loss_ref):
    part = part_ref[:]
    s = jnp.sum(part, axis=1, keepdims=True)
    a = s + b_ref[0, 0]
    logits_ref[:] = a
    n = part.shape[0] // 2
    l0 = a[:n]
    l1 = a[n:]
    m = jnp.maximum(l0, l1)
    logz = m + jnp.log(jnp.exp(l0 - m) + jnp.exp(l1 - m))
    ll = jnp.where(lab_ref[:] == 0, l0, l1)
    loss_ref[0, 0] = jnp.mean(logz - ll)


def kernel(warrant0s, warrant1s, label_ids, emb, W, b):
    B = warrant0s.shape[0]
    bags = 2 * B
    iters = bags // NW // 2

    idx = jnp.concatenate(
        [warrant0s.astype(jnp.int32), warrant1s.astype(jnp.int32)],
        axis=0).reshape(B, 2 * L)

    wck = jnp.pad(W.reshape(-1), (0, NCHUNK * 16 - D)).reshape(NCHUNK, 16)

    emb3 = _seg_table(emb)

    part = _make_sc_call(bags, iters)(idx, emb3, wck)

    logits_col, loss_arr = pl.pallas_call(
        _tc_body,
        out_shape=[
            jax.ShapeDtypeStruct((bags, 1), jnp.float32),
            jax.ShapeDtypeStruct((1, 1), jnp.float32),
        ],
        in_specs=[
            pl.BlockSpec(memory_space=pltpu.VMEM),
            pl.BlockSpec(memory_space=pltpu.VMEM),
            pl.BlockSpec(memory_space=pltpu.SMEM),
        ],
        out_specs=[
            pl.BlockSpec(memory_space=pltpu.VMEM),
            pl.BlockSpec(memory_space=pltpu.SMEM),
        ],
    )(part, label_ids.astype(jnp.int32).reshape(B, 1), b.reshape(1, 1))

    logits = jnp.concatenate([logits_col[:B], logits_col[B:]], axis=1)
    return (loss_arr[0, 0], logits)
